# baseline (speedup 1.0000x reference)
import functools

import jax
import jax.numpy as jnp
from jax import lax
from jax.experimental import pallas as pl
from jax.experimental.pallas import tpu as pltpu
from jax.experimental.pallas import tpu_sc as plsc

M = 4096
R = 20
D = 1000
NCH = 5
CW = D // NCH
NC = 2
NS = 16
NW = NC * NS
S = 4
NP = NW // S
MWS = M // NP
RG = R // S
L = 16
NG = MWS // L


@functools.partial(
    pl.kernel,
    mesh=plsc.VectorSubcoreMesh(core_axis_name="c", subcore_axis_name="s"),
    compiler_params=pltpu.CompilerParams(needs_layout_passes=False),
    out_type=jax.ShapeDtypeStruct((R, D, M), jnp.float32),
    scratch_types=[
        pltpu.VMEM((MWS,), jnp.int32),
        pltpu.VMEM((CW, MWS), jnp.float32),
    ],
)
def _onehot_sc(xt_hbm, z_hbm, out_hbm, idx_v, buf_v):
    cid = lax.axis_index("c")
    sid = lax.axis_index("s")
    wid = sid * NC + cid
    p = wid // S
    h = wid % S
    m0 = p * MWS
    r0 = h * RG
    pltpu.sync_copy(z_hbm, buf_v)

    lanes = lax.iota(jnp.int32, L)
    ones = jnp.full((L,), 1.0, jnp.float32)
    zeros = jnp.zeros((L,), jnp.float32)

    def scatter(lo, val):
        for g in range(NG):
            cols = idx_v[pl.ds(g * L, L)] - lo
            mask = (cols >= 0) & (cols < CW)
            plsc.store_scatter(buf_v, [cols, g * L + lanes], val, mask=mask)

    def slab_body(r, carry):
        pltpu.sync_copy(xt_hbm.at[r, pl.ds(m0, MWS)], idx_v)
        for c in range(NCH):
            scatter(c * CW, ones)
            pltpu.sync_copy(
                buf_v,
                out_hbm.at[r, pl.ds(c * CW, CW), pl.ds(m0, MWS)],
            )
            scatter(c * CW, zeros)
        return carry

    lax.fori_loop(r0, r0 + RG, slab_body, 0)


def kernel(x, eye):
    del eye
    xt = jnp.transpose(x)
    zeros = jnp.zeros((CW, MWS), jnp.float32)
    out = _onehot_sc(xt, zeros)
    return jnp.transpose(out, (2, 0, 1))
